# baseline (speedup 1.0000x reference)
import functools

import jax
import jax.numpy as jnp
from jax import lax
from jax.experimental import pallas as pl
from jax.experimental.pallas import tpu as pltpu
from jax.experimental.pallas import tpu_sc as plsc

HIDDEN = 1024
LANES = 16
NUM_CORES = 2
NUM_SUBCORES = 16
NW = NUM_CORES * NUM_SUBCORES
DRAIN_LAG = 6


ROWS_A = 1072
ROWS_B = 976


def _embed(table, idx_flat):
    n = idx_flat.shape[0]
    pair = ROWS_A + ROWS_B

    mesh = plsc.VectorSubcoreMesh(core_axis_name="c", subcore_axis_name="s")

    @functools.partial(
        pl.kernel,
        out_type=jax.ShapeDtypeStruct((n, HIDDEN), jnp.float32),
        mesh=mesh,
        scratch_types=[
            pltpu.VMEM((ROWS_A,), jnp.int32),
            pltpu.VMEM((2, HIDDEN), jnp.float32),
            pltpu.SemaphoreType.DMA,
        ],
    )
    def k(table_hbm, idx_hbm, out_hbm, idx_v, tab_v, sem):
        cid = lax.axis_index("c")
        sid = lax.axis_index("s")
        base = pl.multiple_of(
            sid * pair + jnp.where(cid == 1, ROWS_B, 0), 16
        )
        count = jnp.where(cid == 1, ROWS_A, ROWS_B)
        idx_cp = pltpu.make_async_copy(
            idx_hbm.at[pl.ds(base, ROWS_A)], idx_v, sem
        )
        tab_cp = pltpu.make_async_copy(table_hbm, tab_v, sem)
        idx_cp.start()
        tab_cp.start()
        idx_cp.wait()
        tab_cp.wait()

        def group_body(g, carry):
            segv = idx_v[pl.ds(g * LANES, LANES)]
            for rr in range(LANES):
                pltpu.make_async_copy(
                    tab_v.at[segv[rr]],
                    out_hbm.at[base + g * LANES + rr],
                    sem,
                ).start()

            @pl.when(g >= DRAIN_LAG)
            def _drain_prev():
                for _ in range(LANES):
                    pltpu.make_async_copy(
                        tab_v.at[0], out_hbm.at[base], sem
                    ).wait()

            return carry

        lax.fori_loop(0, count // LANES, group_body, 0)
        for _ in range(DRAIN_LAG * LANES):
            pltpu.make_async_copy(tab_v.at[0], out_hbm.at[base], sem).wait()

    return k(table, idx_flat)


def kernel(segments, table):
    b, s = segments.shape
    out = _embed(table, segments.reshape(b * s))
    return out.reshape(b, s, HIDDEN)


# speedup vs baseline: 1.0065x; 1.0065x over previous
import functools

import jax
import jax.numpy as jnp
from jax import lax
from jax.experimental import pallas as pl
from jax.experimental.pallas import tpu as pltpu
from jax.experimental.pallas import tpu_sc as plsc

HIDDEN = 1024
LANES = 16
NUM_CORES = 2
NUM_SUBCORES = 16
NW = NUM_CORES * NUM_SUBCORES
DRAIN_LAG = 2


ROWS_A = 1072
ROWS_B = 976


def _embed(table, idx_flat):
    n = idx_flat.shape[0]
    pair = ROWS_A + ROWS_B

    mesh = plsc.VectorSubcoreMesh(core_axis_name="c", subcore_axis_name="s")

    @functools.partial(
        pl.kernel,
        out_type=jax.ShapeDtypeStruct((n, HIDDEN), jnp.float32),
        mesh=mesh,
        scratch_types=[
            pltpu.VMEM((ROWS_A,), jnp.int32),
            pltpu.VMEM((2, HIDDEN), jnp.float32),
            pltpu.SemaphoreType.DMA,
        ],
    )
    def k(table_hbm, idx_hbm, out_hbm, idx_v, tab_v, sem):
        cid = lax.axis_index("c")
        sid = lax.axis_index("s")
        base = pl.multiple_of(
            sid * pair + jnp.where(cid == 1, ROWS_B, 0), 16
        )
        count = jnp.where(cid == 1, ROWS_A, ROWS_B)
        idx_cp = pltpu.make_async_copy(
            idx_hbm.at[pl.ds(base, ROWS_A)], idx_v, sem
        )
        tab_cp = pltpu.make_async_copy(table_hbm, tab_v, sem)
        idx_cp.start()
        tab_cp.start()
        idx_cp.wait()
        tab_cp.wait()

        def group_body(g, carry):
            segv = idx_v[pl.ds(g * LANES, LANES)]
            for rr in range(LANES):
                pltpu.make_async_copy(
                    tab_v.at[segv[rr]],
                    out_hbm.at[base + g * LANES + rr],
                    sem,
                ).start()

            @pl.when(g >= DRAIN_LAG)
            def _drain_prev():
                for _ in range(LANES):
                    pltpu.make_async_copy(
                        tab_v.at[0], out_hbm.at[base], sem
                    ).wait()

            return carry

        lax.fori_loop(0, count // LANES, group_body, 0)
        for _ in range(DRAIN_LAG * LANES):
            pltpu.make_async_copy(tab_v.at[0], out_hbm.at[base], sem).wait()

    return k(table, idx_flat)


def kernel(segments, table):
    b, s = segments.shape
    out = _embed(table, segments.reshape(b * s))
    return out.reshape(b, s, HIDDEN)


# speedup vs baseline: 1.0082x; 1.0017x over previous
import functools

import jax
import jax.numpy as jnp
from jax import lax
from jax.experimental import pallas as pl
from jax.experimental.pallas import tpu as pltpu
from jax.experimental.pallas import tpu_sc as plsc

HIDDEN = 1024
LANES = 16
NUM_CORES = 2
NUM_SUBCORES = 16
NW = NUM_CORES * NUM_SUBCORES
DRAIN_LAG = 3


ROWS_A = 1072
ROWS_B = 976


def _embed(table, idx_flat):
    n = idx_flat.shape[0]
    pair = ROWS_A + ROWS_B

    mesh = plsc.VectorSubcoreMesh(core_axis_name="c", subcore_axis_name="s")

    @functools.partial(
        pl.kernel,
        out_type=jax.ShapeDtypeStruct((n, HIDDEN), jnp.float32),
        mesh=mesh,
        scratch_types=[
            pltpu.VMEM((ROWS_A,), jnp.int32),
            pltpu.VMEM((2, HIDDEN), jnp.float32),
            pltpu.SemaphoreType.DMA,
        ],
    )
    def k(table_hbm, idx_hbm, out_hbm, idx_v, tab_v, sem):
        cid = lax.axis_index("c")
        sid = lax.axis_index("s")
        base = pl.multiple_of(
            sid * pair + jnp.where(cid == 1, ROWS_B, 0), 16
        )
        count = jnp.where(cid == 1, ROWS_A, ROWS_B)
        idx_cp = pltpu.make_async_copy(
            idx_hbm.at[pl.ds(base, ROWS_A)], idx_v, sem
        )
        tab_cp = pltpu.make_async_copy(table_hbm, tab_v, sem)
        idx_cp.start()
        tab_cp.start()
        idx_cp.wait()
        tab_cp.wait()

        def group_body(g, carry):
            segv = idx_v[pl.ds(g * LANES, LANES)]
            for rr in range(LANES):
                pltpu.make_async_copy(
                    tab_v.at[segv[rr]],
                    out_hbm.at[base + g * LANES + rr],
                    sem,
                ).start()

            @pl.when(g >= DRAIN_LAG)
            def _drain_prev():
                for _ in range(LANES):
                    pltpu.make_async_copy(
                        tab_v.at[0], out_hbm.at[base], sem
                    ).wait()

            return carry

        lax.fori_loop(0, count // LANES, group_body, 0)
        for _ in range(DRAIN_LAG * LANES):
            pltpu.make_async_copy(tab_v.at[0], out_hbm.at[base], sem).wait()

    return k(table, idx_flat)


def kernel(segments, table):
    b, s = segments.shape
    out = _embed(table, segments.reshape(b * s))
    return out.reshape(b, s, HIDDEN)


# speedup vs baseline: 1.0098x; 1.0016x over previous
import functools

import jax
import jax.numpy as jnp
from jax import lax
from jax.experimental import pallas as pl
from jax.experimental.pallas import tpu as pltpu
from jax.experimental.pallas import tpu_sc as plsc

HIDDEN = 1024
LANES = 16
DRAIN_LAG = 3
ROWS_A = 1072
ROWS_B = 976


def _embed(table, idx_flat):
    n = idx_flat.shape[0]
    pair = ROWS_A + ROWS_B

    mesh = plsc.VectorSubcoreMesh(core_axis_name="c", subcore_axis_name="s")

    @functools.partial(
        pl.kernel,
        out_type=jax.ShapeDtypeStruct((n, HIDDEN), jnp.float32),
        mesh=mesh,
        scratch_types=[
            pltpu.VMEM((ROWS_A,), jnp.int32),
            pltpu.VMEM((2, HIDDEN), jnp.float32),
            pltpu.SemaphoreType.DMA,
        ],
    )
    def k(table_hbm, idx_hbm, out_hbm, idx_v, tab_v, sem):
        cid = lax.axis_index("c")
        sid = lax.axis_index("s")
        base = pl.multiple_of(
            sid * pair + jnp.where(cid == 1, ROWS_B, 0), 16
        )
        count = jnp.where(cid == 1, ROWS_A, ROWS_B)
        idx_cp = pltpu.make_async_copy(
            idx_hbm.at[pl.ds(base, ROWS_A)], idx_v, sem
        )
        tab_cp = pltpu.make_async_copy(table_hbm, tab_v, sem)
        idx_cp.start()
        tab_cp.start()
        idx_cp.wait()
        tab_cp.wait()

        def group_body(g, carry):
            segv = idx_v[pl.ds(g * LANES, LANES)]
            for rr in range(LANES):
                pltpu.make_async_copy(
                    tab_v.at[segv[rr]],
                    out_hbm.at[base + g * LANES + rr],
                    sem,
                ).start()

            @pl.when(g >= DRAIN_LAG)
            def _drain_prev():
                for _ in range(LANES):
                    pltpu.make_async_copy(
                        tab_v.at[0], out_hbm.at[base], sem
                    ).wait()

            return carry

        lax.fori_loop(0, count // LANES, group_body, 0)
        for _ in range(DRAIN_LAG * LANES):
            pltpu.make_async_copy(tab_v.at[0], out_hbm.at[base], sem).wait()

    return k(table, idx_flat)


def kernel(segments, table):
    b, s = segments.shape
    out = _embed(table, segments.reshape(b * s))
    return out.reshape(b, s, HIDDEN)
